# baseline (speedup 1.0000x reference)
import jax
import jax.numpy as jnp
from jax.experimental import pallas as pl
from jax.experimental.pallas import tpu as pltpu

_TN = 1024
_NCH = 2
_EPS_LOG = 1e-10


def _vq_body(x_ref, cb_ref, out_ref, idx_ref, perp_ref,
             acc_ref, c2_ref, aug_ref):
    m, d = cb_ref.shape[1], cb_ref.shape[2]
    tn = x_ref.shape[2]
    ch = tn // _NCH
    r_stages = idx_ref.shape[0]
    b = pl.program_id(1)
    t = pl.program_id(2)
    nb = pl.num_programs(1)
    nt = pl.num_programs(2)

    @pl.when(jnp.logical_and(b == 0, t == 0))
    def _per_head_setup():
        cb = cb_ref[0]
        norm = jnp.sqrt(jnp.sum(cb * cb, axis=1, keepdims=True))
        c2_ref[...] = (cb / jnp.maximum(norm, 1e-12)).astype(jnp.bfloat16)
        aug_ref[...] = cb.astype(jnp.bfloat16)
        acc_ref[...] = jnp.zeros_like(acc_ref)

    c2 = c2_ref[...]
    aug = aug_ref[...]
    xv = x_ref[0]

    for c in range(_NCH):
        lo, hi = c * ch, (c + 1) * ch
        iota_m = jax.lax.broadcasted_iota(jnp.int32, (m, ch), 0)
        resid = xv[:, lo:hi]
        total = jnp.zeros_like(resid)
        for r in range(r_stages):
            qn = jnp.sqrt(jnp.sum(resid * resid, axis=0, keepdims=True))
            q2 = (resid / jnp.maximum(qn, 1e-12)).astype(jnp.bfloat16)
            sim = jax.lax.dot_general(
                c2, q2, (((1,), (0,)), ((), ())),
                preferred_element_type=jnp.float32)
            mx = jnp.max(sim, axis=0, keepdims=True)
            idx = jnp.min(jnp.where(sim == mx, iota_m, m),
                          axis=0, keepdims=True)
            oh = (iota_m == idx).astype(jnp.bfloat16)
            quant = jax.lax.dot_general(
                aug, oh, (((0,), (0,)), ((), ())),
                preferred_element_type=jnp.float32)
            resid = resid - quant
            total = total + quant
            idx_ref[r, 0, 0, 0, lo:hi] = idx.reshape(ch)
            acc_ref[:, r:r + 1] += jnp.sum(
                oh.astype(jnp.float32), axis=1, keepdims=True)
        out_ref[0, :, lo:hi] = total

    @pl.when(jnp.logical_and(b == nb - 1, t == nt - 1))
    def _finalize():
        mean = acc_ref[...] / (nb * nt * tn)
        ent = -jnp.sum(mean * jnp.log(mean + _EPS_LOG), axis=0, keepdims=True)
        perp_ref[0] = jnp.exp(ent)


def kernel(x, codebooks):
    bsz, chan, tlen = x.shape
    h, m, d = codebooks.shape
    r_stages = 2
    nt = tlen // _TN
    out, idx, perp = pl.pallas_call(
        _vq_body,
        grid=(h, bsz, nt),
        in_specs=[
            pl.BlockSpec((1, d, _TN), lambda hh, bb, tt: (bb, hh, tt)),
            pl.BlockSpec((1, m, d), lambda hh, bb, tt: (hh, 0, 0)),
        ],
        out_specs=[
            pl.BlockSpec((1, d, _TN), lambda hh, bb, tt: (bb, hh, tt)),
            pl.BlockSpec((r_stages, 1, 1, 1, _TN),
                         lambda hh, bb, tt: (0, bb, hh, 0, tt)),
            pl.BlockSpec((1, 1, r_stages), lambda hh, bb, tt: (hh, 0, 0)),
        ],
        out_shape=[
            jax.ShapeDtypeStruct((bsz, chan, tlen), jnp.float32),
            jax.ShapeDtypeStruct((r_stages, bsz, h, 1, tlen), jnp.int32),
            jax.ShapeDtypeStruct((h, 1, r_stages), jnp.float32),
        ],
        scratch_shapes=[
            pltpu.VMEM((m, r_stages), jnp.float32),
            pltpu.VMEM((m, d), jnp.bfloat16),
            pltpu.VMEM((m, d), jnp.bfloat16),
        ],
    )(x, codebooks)
    indices = jnp.transpose(idx.reshape(r_stages, bsz, h, tlen), (1, 2, 3, 0))
    perplexity = perp.reshape(h * r_stages)
    return out, indices, perplexity


# speedup vs baseline: 1.2119x; 1.2119x over previous
import jax
import jax.numpy as jnp
from jax.experimental import pallas as pl
from jax.experimental.pallas import tpu as pltpu

_TN = 2048
_NCH = 1
_EPS_LOG = 1e-10


def _vq_body(x_ref, cb_ref, out_ref, idx_ref, perp_ref,
             acc_ref, c2_ref, aug_ref):
    m, d = cb_ref.shape[1], cb_ref.shape[2]
    tn = x_ref.shape[2]
    ch = tn // _NCH
    r_stages = idx_ref.shape[0]
    b = pl.program_id(1)
    t = pl.program_id(2)
    nb = pl.num_programs(1)
    nt = pl.num_programs(2)

    @pl.when(jnp.logical_and(b == 0, t == 0))
    def _per_head_setup():
        cb = cb_ref[0]
        norm = jnp.sqrt(jnp.sum(cb * cb, axis=1, keepdims=True))
        c2_ref[...] = (cb / jnp.maximum(norm, 1e-12)).astype(jnp.bfloat16)
        aug_ref[...] = cb.astype(jnp.bfloat16)
        acc_ref[...] = jnp.zeros_like(acc_ref)

    c2 = c2_ref[...]
    aug = aug_ref[...]
    xv = x_ref[0]

    for c in range(_NCH):
        lo, hi = c * ch, (c + 1) * ch
        iota_m = jax.lax.broadcasted_iota(jnp.int32, (m, ch), 0)
        resid = xv[:, lo:hi]
        total = jnp.zeros_like(resid)
        for r in range(r_stages):
            qn = jnp.sqrt(jnp.sum(resid * resid, axis=0, keepdims=True))
            q2 = (resid / jnp.maximum(qn, 1e-12)).astype(jnp.bfloat16)
            sim = jax.lax.dot_general(
                c2, q2, (((1,), (0,)), ((), ())),
                preferred_element_type=jnp.float32)
            mx = jnp.max(sim, axis=0, keepdims=True)
            idx = jnp.min(jnp.where(sim == mx, iota_m, m),
                          axis=0, keepdims=True)
            oh = (iota_m == idx).astype(jnp.bfloat16)
            quant = jax.lax.dot_general(
                aug, oh, (((0,), (0,)), ((), ())),
                preferred_element_type=jnp.float32)
            resid = resid - quant
            total = total + quant
            idx_ref[r, 0, 0, 0, lo:hi] = idx.reshape(ch)
            acc_ref[:, r:r + 1] += jnp.sum(
                oh.astype(jnp.float32), axis=1, keepdims=True)
        out_ref[0, :, lo:hi] = total

    @pl.when(jnp.logical_and(b == nb - 1, t == nt - 1))
    def _finalize():
        mean = acc_ref[...] / (nb * nt * tn)
        ent = -jnp.sum(mean * jnp.log(mean + _EPS_LOG), axis=0, keepdims=True)
        perp_ref[0] = jnp.exp(ent)


def kernel(x, codebooks):
    bsz, chan, tlen = x.shape
    h, m, d = codebooks.shape
    r_stages = 2
    nt = tlen // _TN
    out, idx, perp = pl.pallas_call(
        _vq_body,
        grid=(h, bsz, nt),
        in_specs=[
            pl.BlockSpec((1, d, _TN), lambda hh, bb, tt: (bb, hh, tt)),
            pl.BlockSpec((1, m, d), lambda hh, bb, tt: (hh, 0, 0)),
        ],
        out_specs=[
            pl.BlockSpec((1, d, _TN), lambda hh, bb, tt: (bb, hh, tt)),
            pl.BlockSpec((r_stages, 1, 1, 1, _TN),
                         lambda hh, bb, tt: (0, bb, hh, 0, tt)),
            pl.BlockSpec((1, 1, r_stages), lambda hh, bb, tt: (hh, 0, 0)),
        ],
        out_shape=[
            jax.ShapeDtypeStruct((bsz, chan, tlen), jnp.float32),
            jax.ShapeDtypeStruct((r_stages, bsz, h, 1, tlen), jnp.int32),
            jax.ShapeDtypeStruct((h, 1, r_stages), jnp.float32),
        ],
        scratch_shapes=[
            pltpu.VMEM((m, r_stages), jnp.float32),
            pltpu.VMEM((m, d), jnp.bfloat16),
            pltpu.VMEM((m, d), jnp.bfloat16),
        ],
    )(x, codebooks)
    indices = jnp.transpose(idx.reshape(r_stages, bsz, h, tlen), (1, 2, 3, 0))
    perplexity = perp.reshape(h * r_stages)
    return out, indices, perplexity


# speedup vs baseline: 1.6119x; 1.3301x over previous
import jax
import jax.numpy as jnp
from jax.experimental import pallas as pl
from jax.experimental.pallas import tpu as pltpu

_TN = 2048
_EPS_LOG = 1e-10


def _vq_body(x_ref, cb_ref, out_ref, idx_ref, perp_ref,
             acc_ref, c2_ref, aug_ref, quant_ref):
    m, d = cb_ref.shape[1], cb_ref.shape[2]
    tn = x_ref.shape[2]
    r_stages = idx_ref.shape[0]
    b = pl.program_id(1)
    t = pl.program_id(2)
    nb = pl.num_programs(1)
    nt = pl.num_programs(2)

    @pl.when(jnp.logical_and(b == 0, t == 0))
    def _per_head_setup():
        cb = cb_ref[0]
        norm = jnp.sqrt(jnp.sum(cb * cb, axis=1, keepdims=True))
        c2_ref[...] = (cb / jnp.maximum(norm, 1e-12)).astype(jnp.bfloat16)
        iota_col = jax.lax.broadcasted_iota(jnp.int32, (m, 1), 0)
        aug_ref[:, 0:d] = cb.astype(jnp.bfloat16)
        aug_ref[:, d:d + 1] = (iota_col // 8).astype(jnp.bfloat16)
        aug_ref[:, d + 1:d + 2] = (iota_col % 8).astype(jnp.bfloat16)
        aug_ref[:, d + 2:d + 3] = jnp.ones((m, 1), jnp.bfloat16)
        aug_ref[:, d + 3:] = jnp.zeros((m, 2 * d - 3 - d), jnp.bfloat16)
        acc_ref[...] = jnp.zeros_like(acc_ref)

    c2 = c2_ref[...]
    aug = aug_ref[...]
    xv = x_ref[0]

    resid = xv
    total = jnp.zeros_like(xv)
    for r in range(r_stages):
        qn = jnp.sqrt(jnp.sum(resid * resid, axis=0, keepdims=True))
        q2 = (resid / jnp.maximum(qn, 1e-12)).astype(jnp.bfloat16)
        sim = jax.lax.dot_general(
            c2, q2, (((1,), (0,)), ((), ())),
            preferred_element_type=jnp.float32)
        mx = jnp.max(sim, axis=0, keepdims=True)
        oh = (sim == mx).astype(jnp.bfloat16)
        gath = jax.lax.dot_general(
            aug, oh, (((0,), (0,)), ((), ())),
            preferred_element_type=jnp.float32)
        idx_fast = (gath[d:d + 1] * 8.0 + gath[d + 1:d + 2]).astype(jnp.int32)
        cnt_fast = jnp.sum(oh.astype(jnp.float32), axis=1, keepdims=True)
        quant_ref[...] = gath[0:d]
        idx_ref[r, 0, 0, 0, :] = idx_fast.reshape(tn)
        acc_ref[:, r:r + 1] += cnt_fast
        ties = jnp.any(gath[d + 2:d + 3] != 1.0)

        @pl.when(ties)
        def _tie_fix():
            iota_m = jax.lax.broadcasted_iota(jnp.int32, (m, tn), 0)
            idxe = jnp.min(jnp.where(sim == mx, iota_m, m),
                           axis=0, keepdims=True)
            oh2 = (iota_m == idxe).astype(jnp.bfloat16)
            gath2 = jax.lax.dot_general(
                aug, oh2, (((0,), (0,)), ((), ())),
                preferred_element_type=jnp.float32)
            quant_ref[...] = gath2[0:d]
            idx_ref[r, 0, 0, 0, :] = idxe.reshape(tn)
            acc_ref[:, r:r + 1] += jnp.sum(
                oh2.astype(jnp.float32), axis=1, keepdims=True) - cnt_fast

        quant = quant_ref[...]
        resid = resid - quant
        total = total + quant
    out_ref[0] = total

    @pl.when(jnp.logical_and(b == nb - 1, t == nt - 1))
    def _finalize():
        mean = acc_ref[...] / (nb * nt * tn)
        ent = -jnp.sum(mean * jnp.log(mean + _EPS_LOG), axis=0, keepdims=True)
        perp_ref[0] = jnp.exp(ent)


def kernel(x, codebooks):
    bsz, chan, tlen = x.shape
    h, m, d = codebooks.shape
    r_stages = 2
    nt = tlen // _TN
    out, idx, perp = pl.pallas_call(
        _vq_body,
        grid=(h, bsz, nt),
        in_specs=[
            pl.BlockSpec((1, d, _TN), lambda hh, bb, tt: (bb, hh, tt)),
            pl.BlockSpec((1, m, d), lambda hh, bb, tt: (hh, 0, 0)),
        ],
        out_specs=[
            pl.BlockSpec((1, d, _TN), lambda hh, bb, tt: (bb, hh, tt)),
            pl.BlockSpec((r_stages, 1, 1, 1, _TN),
                         lambda hh, bb, tt: (0, bb, hh, 0, tt)),
            pl.BlockSpec((1, 1, r_stages), lambda hh, bb, tt: (hh, 0, 0)),
        ],
        out_shape=[
            jax.ShapeDtypeStruct((bsz, chan, tlen), jnp.float32),
            jax.ShapeDtypeStruct((r_stages, bsz, h, 1, tlen), jnp.int32),
            jax.ShapeDtypeStruct((h, 1, r_stages), jnp.float32),
        ],
        scratch_shapes=[
            pltpu.VMEM((m, r_stages), jnp.float32),
            pltpu.VMEM((m, d), jnp.bfloat16),
            pltpu.VMEM((m, 2 * d), jnp.bfloat16),
            pltpu.VMEM((d, _TN), jnp.float32),
        ],
    )(x, codebooks)
    indices = jnp.transpose(idx.reshape(r_stages, bsz, h, tlen), (1, 2, 3, 0))
    perplexity = perp.reshape(h * r_stages)
    return out, indices, perplexity


# speedup vs baseline: 1.6260x; 1.0087x over previous
import jax
import jax.numpy as jnp
from jax.experimental import pallas as pl
from jax.experimental.pallas import tpu as pltpu

_TN = 2048
_EPS_LOG = 1e-10


def _vq_body(x_ref, cb_ref, out_ref, idx_ref, perp_ref,
             acc_ref, c2_ref, aug_ref):
    m, d = cb_ref.shape[1], cb_ref.shape[2]
    tn = x_ref.shape[2]
    r_stages = idx_ref.shape[0]
    b = pl.program_id(1)
    t = pl.program_id(2)
    nb = pl.num_programs(1)
    nt = pl.num_programs(2)

    @pl.when(jnp.logical_and(b == 0, t == 0))
    def _per_head_setup():
        cb = cb_ref[0]
        norm = jnp.sqrt(jnp.sum(cb * cb, axis=1, keepdims=True))
        c2_ref[...] = (cb / jnp.maximum(norm, 1e-12)).astype(jnp.bfloat16)
        iota_col = jax.lax.broadcasted_iota(jnp.int32, (m, 1), 0)
        aug_ref[:, 0:d] = cb.astype(jnp.bfloat16)
        aug_ref[:, d:d + 1] = (iota_col // 8).astype(jnp.bfloat16)
        aug_ref[:, d + 1:d + 2] = (iota_col % 8).astype(jnp.bfloat16)
        aug_ref[:, d + 2:d + 3] = jnp.ones((m, 1), jnp.bfloat16)
        aug_ref[:, d + 3:] = jnp.zeros((m, 2 * d - 3 - d), jnp.bfloat16)
        acc_ref[...] = jnp.zeros_like(acc_ref)

    c2 = c2_ref[...]
    aug = aug_ref[...]
    xv = x_ref[0]

    def _stage(resid):
        qn = jnp.sqrt(jnp.sum(resid * resid, axis=0, keepdims=True))
        q2 = (resid / jnp.maximum(qn, 1e-12)).astype(jnp.bfloat16)
        sim = jax.lax.dot_general(
            c2, q2, (((1,), (0,)), ((), ())),
            preferred_element_type=jnp.float32)
        mx = jnp.max(sim, axis=0, keepdims=True)
        oh = (sim == mx).astype(jnp.bfloat16)
        gath = jax.lax.dot_general(
            aug, oh, (((0,), (0,)), ((), ())),
            preferred_element_type=jnp.float32)
        return sim, mx, oh, gath

    def _exact_idx(sim, mx):
        iota_m = jax.lax.broadcasted_iota(jnp.int32, (m, tn), 0)
        idxe = jnp.min(jnp.where(sim == mx, iota_m, m),
                       axis=0, keepdims=True)
        oh2 = (iota_m == idxe).astype(jnp.bfloat16)
        return idxe, oh2

    resid = xv
    total = jnp.zeros_like(xv)
    ties = []
    for r in range(r_stages):
        sim, mx, oh, gath = _stage(resid)
        idx_fast = (gath[d:d + 1] * 8.0 + gath[d + 1:d + 2]).astype(jnp.int32)
        idx_ref[r, 0, 0, 0, :] = idx_fast.reshape(tn)
        acc_ref[:, r:r + 1] += jnp.sum(
            oh.astype(jnp.float32), axis=1, keepdims=True)
        ties.append(gath[d + 2:d + 3])
        quant = gath[0:d]
        resid = resid - quant
        total = total + quant
    out_ref[0] = total

    tie_any = jnp.any((ties[0] != 1.0) | (ties[1] != 1.0))

    @pl.when(tie_any)
    def _tie_fix():
        resid_f = xv
        resid_e = xv
        total_e = jnp.zeros_like(xv)
        for r in range(r_stages):
            sim_f, mx_f, oh_f, gath_f = _stage(resid_f)
            acc_ref[:, r:r + 1] -= jnp.sum(
                oh_f.astype(jnp.float32), axis=1, keepdims=True)
            resid_f = resid_f - gath_f[0:d]
            sim_e, mx_e, oh_e0, gath_e0 = _stage(resid_e)
            idxe, oh_e = _exact_idx(sim_e, mx_e)
            quant_e = jax.lax.dot_general(
                aug, oh_e, (((0,), (0,)), ((), ())),
                preferred_element_type=jnp.float32)[0:d]
            idx_ref[r, 0, 0, 0, :] = idxe.reshape(tn)
            acc_ref[:, r:r + 1] += jnp.sum(
                oh_e.astype(jnp.float32), axis=1, keepdims=True)
            resid_e = resid_e - quant_e
            total_e = total_e + quant_e
        out_ref[0] = total_e

    @pl.when(jnp.logical_and(b == nb - 1, t == nt - 1))
    def _finalize():
        mean = acc_ref[...] / (nb * nt * tn)
        ent = -jnp.sum(mean * jnp.log(mean + _EPS_LOG), axis=0, keepdims=True)
        perp_ref[0] = jnp.exp(ent)


def kernel(x, codebooks):
    bsz, chan, tlen = x.shape
    h, m, d = codebooks.shape
    r_stages = 2
    nt = tlen // _TN
    out, idx, perp = pl.pallas_call(
        _vq_body,
        grid=(h, bsz, nt),
        in_specs=[
            pl.BlockSpec((1, d, _TN), lambda hh, bb, tt: (bb, hh, tt)),
            pl.BlockSpec((1, m, d), lambda hh, bb, tt: (hh, 0, 0)),
        ],
        out_specs=[
            pl.BlockSpec((1, d, _TN), lambda hh, bb, tt: (bb, hh, tt)),
            pl.BlockSpec((r_stages, 1, 1, 1, _TN),
                         lambda hh, bb, tt: (0, bb, hh, 0, tt)),
            pl.BlockSpec((1, 1, r_stages), lambda hh, bb, tt: (hh, 0, 0)),
        ],
        out_shape=[
            jax.ShapeDtypeStruct((bsz, chan, tlen), jnp.float32),
            jax.ShapeDtypeStruct((r_stages, bsz, h, 1, tlen), jnp.int32),
            jax.ShapeDtypeStruct((h, 1, r_stages), jnp.float32),
        ],
        scratch_shapes=[
            pltpu.VMEM((m, r_stages), jnp.float32),
            pltpu.VMEM((m, d), jnp.bfloat16),
            pltpu.VMEM((m, 2 * d), jnp.bfloat16),
        ],
    )(x, codebooks)
    indices = jnp.transpose(idx.reshape(r_stages, bsz, h, tlen), (1, 2, 3, 0))
    perplexity = perp.reshape(h * r_stages)
    return out, indices, perplexity
